# speedup vs baseline: 1.9707x; 1.9707x over previous
import functools

import jax
import jax.numpy as jnp
from jax import lax
from jax.experimental import pallas as pl
from jax.experimental.pallas import tpu as pltpu

M = 2048
H = 1024
I = 768
E = 8
BM = 256


def _moe_dense_body(x_ref, gate_ref, w13_ref, w2_ref, out_ref):
    x = x_ref[...]
    gate_w = gate_ref[...]
    logits = lax.dot_general(x, gate_w, (((1,), (1,)), ((), ())),
                             preferred_element_type=jnp.float32)
    idx = lax.broadcasted_iota(jnp.int32, (BM, E), 1)
    m1 = jnp.max(logits, axis=1, keepdims=True)
    is1 = logits == m1
    id1 = jnp.min(jnp.where(is1, idx, E), axis=1, keepdims=True)
    masked = jnp.where(idx == id1, -jnp.inf, logits)
    m2 = jnp.max(masked, axis=1, keepdims=True)
    is2 = masked == m2
    id2 = jnp.min(jnp.where(is2, idx, E), axis=1, keepdims=True)
    r = jnp.exp(m2 - m1)
    t1 = 1.0 / (1.0 + r)
    t2 = r / (1.0 + r)

    xb = x.astype(jnp.bfloat16)
    acc = jnp.zeros((BM, H), dtype=jnp.float32)
    for e in range(E):
        ce = t1[:, 0] * (id1[:, 0] == e) + t2[:, 0] * (id2[:, 0] == e)
        gu = lax.dot_general(xb, w13_ref[e], (((1,), (1,)), ((), ())),
                             preferred_element_type=jnp.float32)
        g = gu[:, :I]
        u = gu[:, I:]
        h = (g * (1.0 / (1.0 + jnp.exp(-g)))) * u
        h = h * ce[:, None]
        y = lax.dot_general(h.astype(jnp.bfloat16), w2_ref[e],
                            (((1,), (1,)), ((), ())),
                            preferred_element_type=jnp.float32)
        acc = acc + y
    out_ref[...] = acc


@jax.jit
def kernel(hidden_states, gate_w, w13, w2):
    x = hidden_states.reshape(-1, H)
    w13b = w13.astype(jnp.bfloat16)
    w2b = w2.astype(jnp.bfloat16)
    out = pl.pallas_call(
        _moe_dense_body,
        grid=(M // BM,),
        in_specs=[
            pl.BlockSpec((BM, H), lambda i: (i, 0)),
            pl.BlockSpec((E, H), lambda i: (0, 0)),
            pl.BlockSpec((E, 2 * I, H), lambda i: (0, 0, 0)),
            pl.BlockSpec((E, H, I), lambda i: (0, 0, 0)),
        ],
        out_specs=pl.BlockSpec((BM, H), lambda i: (i, 0)),
        out_shape=jax.ShapeDtypeStruct((M, H), jnp.float32),
        compiler_params=pltpu.CompilerParams(
            vmem_limit_bytes=100 * 1024 * 1024,
        ),
    )(x, gate_w, w13b, w2b)
    return out.reshape(hidden_states.shape)
